# baseline (speedup 1.0000x reference)
import functools

import jax
import jax.numpy as jnp
from jax import lax
from jax.experimental import pallas as pl
from jax.experimental.pallas import tpu as pltpu
from jax.experimental.pallas import tpu_sc as plsc

N = 10000
E = 160000
H = 256
HH = 128
NL = 4
OUT = 128
G = 256

NC = 2
NS = 16
HALF = N // 2
K = 32
NCH = 168
EPT = K * NCH
ES = NS * EPT
RZ = 80
NRC = N // RZ
ZR = 40
NZC = HALF // ZR
AGG_R = 5008
MB = 400
NBLK = N // MB

_mesh = plsc.VectorSubcoreMesh(core_axis_name="c", subcore_axis_name="s",
                               num_cores=NC, num_subcores=NS)


def _zero_vmem(buf, rows, cols):
    z = jnp.zeros((16,), jnp.float32)

    def body(r, _):
        for v in range(cols // 16):
            buf[r, pl.ds(v * 16, 16)] = z
        return 0

    lax.fori_loop(0, rows, body, 0)


@functools.partial(
    pl.kernel,
    out_type=(jax.ShapeDtypeStruct((N, HH), jnp.float32),
              jax.ShapeDtypeStruct((N, HH), jnp.float32)),
    mesh=_mesh,
    scratch_types=[
        pltpu.VMEM((9, RZ), jnp.int32),
    ] + [pltpu.VMEM((RZ, HH), jnp.float32) for _ in range(9)] + [
        pltpu.SemaphoreType.DMA,
    ],
)
def _atom_kernel(xoffr_hbm, acat_hbm, h0_hbm, h1_hbm, idxb,
                 b0, b1, b2, b3, b4, b5, b6, b7, b8, sem):
    cid = lax.axis_index("c")
    sid = lax.axis_index("s")
    toff = cid * 900
    bufs = (b0, b1, b2, b3, b4, b5, b6, b7, b8)

    def chunk(ch):
        base = ch * RZ
        pltpu.sync_copy(xoffr_hbm.at[ch], idxb)
        for j in range(9):
            for t in range(RZ // 16):
                s = pl.ds(t * 16, 16)
                idxb[j, s] = idxb[j, s] + toff
        for j in range(9):
            pltpu.async_copy(acat_hbm.at[idxb.at[j]], bufs[j], sem)
        for j in range(9):
            pltpu.make_async_copy(acat_hbm.at[idxb.at[j]], bufs[j], sem).wait()

        def rowbody(r, _):
            for v in range(HH // 16):
                s = pl.ds(v * 16, 16)
                acc = bufs[0][r, s]
                for j in range(1, 9):
                    acc = acc + bufs[j][r, s]
                bufs[0][r, s] = acc
            return 0

        lax.fori_loop(0, RZ, rowbody, 0)

        @pl.when(cid == 0)
        def _():
            pltpu.sync_copy(bufs[0], h0_hbm.at[pl.ds(base, RZ)])

        @pl.when(cid == 1)
        def _():
            pltpu.sync_copy(bufs[0], h1_hbm.at[pl.ds(base, RZ)])

    for k in range(8):
        ch = sid + NS * k

        @pl.when(ch < NRC)
        def _():
            chunk(ch)


@functools.partial(
    pl.kernel,
    out_type=jax.ShapeDtypeStruct((N, H), jnp.float32),
    mesh=_mesh,
    scratch_types=(
        [pltpu.VMEM((2, K), jnp.int32) for _ in range(4)]
        + [pltpu.VMEM((K,), jnp.int32) for _ in range(4)]
        + [pltpu.VMEM((K, H), jnp.float32) for _ in range(4)]
        + [pltpu.VMEM((K, HH), jnp.float32) for _ in range(2)]
        + [
            pltpu.VMEM((ZR, HH), jnp.float32),
            pltpu.VMEM_SHARED((AGG_R, HH), jnp.float32),
            pltpu.VMEM_SHARED((AGG_R, HH), jnp.float32),
        ]
        + [pltpu.SemaphoreType.DMA for _ in range(6)]
    ),
)
def _edge_kernel(eidx_hbm, dstp_hbm, h_hbm, ccat_hbm, agg_hbm,
                 ib0, ib1, ib2, ib3, db0, db1, db2, db3,
                 hbuf0, hbuf1, cbuf0, cbuf1, mbufA, mbufB, zrows, aggA, aggB,
                 semh0, semh1, semc0, semc1, semx0, semx1):
    cid = lax.axis_index("c")
    sid = lax.axis_index("s")
    ib = (ib0, ib1, ib2, ib3)
    db = (db0, db1, db2, db3)
    hbuf = (hbuf0, hbuf1)
    cbuf = (cbuf0, cbuf1)
    semh = (semh0, semh1)
    semc = (semc0, semc1)
    semx = (semx0, semx1)

    _zero_vmem(zrows, ZR, HH)
    for k in range(8):
        ch = sid + NS * k

        @pl.when(ch < NZC)
        def _():
            pltpu.sync_copy(zrows, aggA.at[pl.ds(ch * ZR, ZR)])
            pltpu.sync_copy(zrows, aggB.at[pl.ds(ch * ZR, ZR)])

    plsc.subcore_barrier()

    cbase = (cid * NS + sid) * NCH
    dbase = (cid * NS + sid) * EPT

    def idxfetch(i, q, x):
        pltpu.async_copy(eidx_hbm.at[cbase + i], ib[q], semx[x])
        pltpu.async_copy(dstp_hbm.at[pl.ds(dbase + i * K, K)], db[q], semx[x])

    def idxwait(q, x):
        pltpu.make_async_copy(eidx_hbm.at[0], ib[q], semx[x]).wait()
        pltpu.make_async_copy(dstp_hbm.at[pl.ds(0, K)], db[q], semx[x]).wait()

    def gissue(q, b):
        pltpu.async_copy(ccat_hbm.at[ib[q].at[1]], cbuf[b], semc[b])
        pltpu.async_copy(h_hbm.at[ib[q].at[0]], hbuf[b], semh[b])

    idxfetch(0, 0, 0)
    idxfetch(1, 1, 1)
    idxwait(0, 0)
    gissue(0, 0)
    idxwait(1, 1)
    gissue(1, 1)
    idxfetch(2, 2, 0)
    idxfetch(3, 3, 1)

    def halfstep(i, q, b):
        pltpu.make_async_copy(h_hbm.at[ib[q].at[0]], hbuf[b], semh[b]).wait()
        pltpu.make_async_copy(h_hbm.at[ib[q].at[0]], cbuf[b], semc[b]).wait()

        def rowbody(r, _):
            for v in range(H // 16):
                s = pl.ds(v * 16, 16)
                m = jnp.maximum(hbuf[b][r, s] + cbuf[b][r, s], 0.0)
                if v < HH // 16:
                    mbufA[r, s] = m
                else:
                    mbufB[r, pl.ds(v * 16 - HH, 16)] = m
            return 0

        lax.fori_loop(0, K, rowbody, 0)
        pltpu.sync_copy(mbufA, aggA.at[db[q]], add=True)
        pltpu.sync_copy(mbufB, aggB.at[db[q]], add=True)

        @pl.when(i + 2 < NCH)
        def _():
            idxwait((q + 2) % 4, b)
            gissue((q + 2) % 4, b)

        @pl.when(i + 4 < NCH)
        def _():
            idxfetch(i + 4, q, b)

    def pipestep(g, _):
        for j in range(4):
            halfstep(4 * g + j, j, j % 2)
        return 0

    lax.fori_loop(0, NCH // 4, pipestep, 0)
    plsc.subcore_barrier()

    hbase = cid * HALF
    for k in range(8):
        ch = sid + NS * k

        @pl.when(ch < NZC)
        def _():
            pltpu.sync_copy(aggA.at[pl.ds(ch * ZR, ZR)],
                            agg_hbm.at[pl.ds(hbase + ch * ZR, ZR), pl.ds(0, HH)])
            pltpu.sync_copy(aggB.at[pl.ds(ch * ZR, ZR)],
                            agg_hbm.at[pl.ds(hbase + ch * ZR, ZR), pl.ds(HH, HH)])


def _mlp_body(hr, ar, w1r, b1r, w2r, b2r, z2r, sumr, sqr):
    i = pl.program_id(0)
    z = hr[...] + ar[...]
    z1 = jnp.maximum(
        jnp.dot(z, w1r[...], preferred_element_type=jnp.float32) + b1r[...], 0.0)
    z2 = jnp.dot(z1, w2r[...], preferred_element_type=jnp.float32) + b2r[...]
    z2r[...] = z2
    p = z2.reshape(MB // 8, 8, H)
    s8 = jnp.sum(p, axis=0)
    q8 = jnp.sum(p * p, axis=0)

    @pl.when(i == 0)
    def _():
        sumr[...] = s8
        sqr[...] = q8

    @pl.when(i != 0)
    def _():
        sumr[...] = sumr[...] + s8
        sqr[...] = sqr[...] + q8


_mlp_call = pl.pallas_call(
    _mlp_body,
    grid=(NBLK,),
    in_specs=[
        pl.BlockSpec((MB, H), lambda i: (i, 0)),
        pl.BlockSpec((MB, H), lambda i: (i, 0)),
        pl.BlockSpec((H, H), lambda i: (0, 0)),
        pl.BlockSpec((1, H), lambda i: (0, 0)),
        pl.BlockSpec((H, H), lambda i: (0, 0)),
        pl.BlockSpec((1, H), lambda i: (0, 0)),
    ],
    out_specs=[
        pl.BlockSpec((MB, H), lambda i: (i, 0)),
        pl.BlockSpec((8, H), lambda i: (0, 0)),
        pl.BlockSpec((8, H), lambda i: (0, 0)),
    ],
    out_shape=[
        jax.ShapeDtypeStruct((N, H), jnp.float32),
        jax.ShapeDtypeStruct((8, H), jnp.float32),
        jax.ShapeDtypeStruct((8, H), jnp.float32),
    ],
)


def _bnpool_body(z2r, sumr, sqr, ohr, gsumr, cntr):
    i = pl.program_id(0)
    mu = jnp.sum(sumr[...], axis=0, keepdims=True) * (1.0 / N)
    ex2 = jnp.sum(sqr[...], axis=0, keepdims=True) * (1.0 / N)
    var = ex2 - mu * mu
    inv = lax.rsqrt(var + 1e-5)
    hh = jnp.maximum((z2r[...] - mu) * inv, 0.0)
    oh = ohr[...]
    gs = lax.dot_general(oh, hh, (((0,), (0,)), ((), ())),
                         precision=lax.Precision.HIGHEST,
                         preferred_element_type=jnp.float32)
    c8 = jnp.sum(oh.reshape(MB // 8, 8, G), axis=0)

    @pl.when(i == 0)
    def _():
        gsumr[...] = gs
        cntr[...] = c8

    @pl.when(i != 0)
    def _():
        gsumr[...] = gsumr[...] + gs
        cntr[...] = cntr[...] + c8


_bnpool_call = pl.pallas_call(
    _bnpool_body,
    grid=(NBLK,),
    in_specs=[
        pl.BlockSpec((MB, H), lambda i: (i, 0)),
        pl.BlockSpec((8, H), lambda i: (0, 0)),
        pl.BlockSpec((8, H), lambda i: (0, 0)),
        pl.BlockSpec((MB, G), lambda i: (i, 0)),
    ],
    out_specs=[
        pl.BlockSpec((G, H), lambda i: (0, 0)),
        pl.BlockSpec((8, G), lambda i: (0, 0)),
    ],
    out_shape=[
        jax.ShapeDtypeStruct((G, H), jnp.float32),
        jax.ShapeDtypeStruct((8, G), jnp.float32),
    ],
)


def _bn_body(z2r, sumr, sqr, hor):
    mu = jnp.sum(sumr[...], axis=0, keepdims=True) * (1.0 / N)
    ex2 = jnp.sum(sqr[...], axis=0, keepdims=True) * (1.0 / N)
    var = ex2 - mu * mu
    inv = lax.rsqrt(var + 1e-5)
    hor[...] = jnp.maximum((z2r[...] - mu) * inv, 0.0)


_bn_call = pl.pallas_call(
    _bn_body,
    grid=(NBLK,),
    in_specs=[
        pl.BlockSpec((MB, H), lambda i: (i, 0)),
        pl.BlockSpec((8, H), lambda i: (0, 0)),
        pl.BlockSpec((8, H), lambda i: (0, 0)),
    ],
    out_specs=pl.BlockSpec((MB, H), lambda i: (i, 0)),
    out_shape=jax.ShapeDtypeStruct((N, H), jnp.float32),
)


def _out_body(gsr, cntr, wo1r, bo1r, wo2r, bo2r, outr):
    c = jnp.maximum(jnp.sum(cntr[...], axis=0, keepdims=True), 1.0)
    g = gsr[...] / c.T
    q = jnp.maximum(
        jnp.dot(g, wo1r[...], preferred_element_type=jnp.float32) + bo1r[...], 0.0)
    outr[...] = jnp.dot(q, wo2r[...], preferred_element_type=jnp.float32) + bo2r[...]


_out_call = pl.pallas_call(
    _out_body,
    grid=(1,),
    in_specs=[
        pl.BlockSpec((G, H), lambda i: (0, 0)),
        pl.BlockSpec((8, G), lambda i: (0, 0)),
        pl.BlockSpec((H, H), lambda i: (0, 0)),
        pl.BlockSpec((1, H), lambda i: (0, 0)),
        pl.BlockSpec((H, OUT), lambda i: (0, 0)),
        pl.BlockSpec((1, OUT), lambda i: (0, 0)),
    ],
    out_specs=pl.BlockSpec((G, OUT), lambda i: (0, 0)),
    out_shape=jax.ShapeDtypeStruct((G, OUT), jnp.float32),
)


def kernel(x, edge_index, edge_attr, batch, atom_emb, bond_emb,
           W1, b1, W2, b2, Wo1, bo1, Wo2, bo2):
    x = x.astype(jnp.int32)
    xoff = (x.T + (jnp.arange(9, dtype=jnp.int32) * 100)[:, None])
    xoffr = xoff.reshape(9, NRC, RZ).transpose(1, 0, 2)
    acat = jnp.concatenate(
        [atom_emb[:, :, :HH].reshape(900, HH),
         atom_emb[:, :, HH:].reshape(900, HH)], axis=0)

    src = edge_index[0].astype(jnp.int32)
    dst = edge_index[1].astype(jnp.int32)
    ea = edge_attr.astype(jnp.int32)
    cidx = ea[:, 0] * 36 + ea[:, 1] * 6 + ea[:, 2]

    key = (dst >= HALF).astype(jnp.int32)
    order = jnp.argsort(key, stable=True)
    count0 = E - jnp.sum(key)
    srcs = jnp.concatenate([src[order], jnp.zeros((ES,), jnp.int32)])
    dsts = jnp.concatenate([dst[order], jnp.zeros((ES,), jnp.int32)])
    cs = jnp.concatenate([cidx[order], jnp.zeros((ES,), jnp.int32)])
    pos = jnp.arange(ES, dtype=jnp.int32)

    def _side(start, nvalid):
        s0 = lax.dynamic_slice(srcs, (start,), (ES,))
        d0 = lax.dynamic_slice(dsts, (start,), (ES,))
        c0 = lax.dynamic_slice(cs, (start,), (ES,))
        valid = pos < nvalid
        s0 = jnp.where(valid, s0, 0)
        d0 = jnp.where(valid, d0 % HALF, HALF)
        c0 = jnp.where(valid, c0, 0)
        sc = jnp.stack([s0.reshape(NS, NCH, K), c0.reshape(NS, NCH, K)], axis=2)
        return sc, d0

    sc0, d0 = _side(jnp.int32(0), count0)
    sc1, d1 = _side(count0, E - count0)
    eidx = jnp.concatenate([sc0, sc1], axis=0).reshape(2 * NS * NCH, 2, K)
    dstp = jnp.concatenate([d0, d1])

    ccat = (bond_emb[:, 0, :, None, None, :]
            + bond_emb[:, 1, None, :, None, :]
            + bond_emb[:, 2, None, None, :, :]).reshape(NL, 216, H)

    batch_oh = (batch.astype(jnp.int32)[:, None]
                == jnp.arange(G, dtype=jnp.int32)[None, :]).astype(jnp.float32)

    h0, h1 = _atom_kernel(xoffr, acat)
    h = jnp.concatenate([h0, h1], axis=1)
    for l in range(NL):
        agg = _edge_kernel(eidx, dstp, h, ccat[l])
        z2, s8, q8 = _mlp_call(h, agg, W1[l],
                               b1[l].reshape(1, H), W2[l], b2[l].reshape(1, H))
        if l < NL - 1:
            h = _bn_call(z2, s8, q8)
        else:
            gsum, cnt8 = _bnpool_call(z2, s8, q8, batch_oh)
    out = _out_call(gsum, cnt8, Wo1, bo1.reshape(1, H),
                    Wo2, bo2.reshape(1, OUT))
    return out


# speedup vs baseline: 1.8855x; 1.8855x over previous
import functools

import jax
import jax.numpy as jnp
from jax import lax
from jax.experimental import pallas as pl
from jax.experimental.pallas import tpu as pltpu
from jax.experimental.pallas import tpu_sc as plsc

N = 10000
E = 160000
H = 256
HH = 128
NL = 4
OUT = 128
G = 256

NC = 2
NS = 16
EPT = 10208
EPAD = NS * EPT
K = 88
NCH = 116
RZ = 80
NRC = N // RZ
ZR = 40
NZC = N // ZR
AGG_R = 10008
MB = 400
NBLK = N // MB

_mesh = plsc.VectorSubcoreMesh(core_axis_name="c", subcore_axis_name="s",
                               num_cores=NC, num_subcores=NS)


def _zero_vmem(buf, rows):
    z = jnp.zeros((16,), jnp.float32)

    def body(r, _):
        for v in range(HH // 16):
            buf[r, pl.ds(v * 16, 16)] = z
        return 0

    lax.fori_loop(0, rows, body, 0)


@functools.partial(
    pl.kernel,
    out_type=(jax.ShapeDtypeStruct((N, HH), jnp.float32),
              jax.ShapeDtypeStruct((N, HH), jnp.float32)),
    mesh=_mesh,
    scratch_types=[
        pltpu.VMEM((9, RZ), jnp.int32),
    ] + [pltpu.VMEM((RZ, HH), jnp.float32) for _ in range(9)] + [
        pltpu.SemaphoreType.DMA,
    ],
)
def _atom_kernel(xoffr_hbm, acat_hbm, h0_hbm, h1_hbm, idxb,
                 b0, b1, b2, b3, b4, b5, b6, b7, b8, sem):
    cid = lax.axis_index("c")
    sid = lax.axis_index("s")
    toff = cid * 900
    bufs = (b0, b1, b2, b3, b4, b5, b6, b7, b8)

    def chunk(ch):
        base = ch * RZ
        pltpu.sync_copy(xoffr_hbm.at[ch], idxb)
        for j in range(9):
            for t in range(RZ // 16):
                s = pl.ds(t * 16, 16)
                idxb[j, s] = idxb[j, s] + toff
        for j in range(9):
            pltpu.async_copy(acat_hbm.at[idxb.at[j]], bufs[j], sem)
        for j in range(9):
            pltpu.make_async_copy(acat_hbm.at[idxb.at[j]], bufs[j], sem).wait()

        def rowbody(r, _):
            for v in range(HH // 16):
                s = pl.ds(v * 16, 16)
                acc = bufs[0][r, s]
                for j in range(1, 9):
                    acc = acc + bufs[j][r, s]
                bufs[0][r, s] = acc
            return 0

        lax.fori_loop(0, RZ, rowbody, 0)

        @pl.when(cid == 0)
        def _():
            pltpu.sync_copy(bufs[0], h0_hbm.at[pl.ds(base, RZ)])

        @pl.when(cid == 1)
        def _():
            pltpu.sync_copy(bufs[0], h1_hbm.at[pl.ds(base, RZ)])

    for k in range(8):
        ch = sid + NS * k

        @pl.when(ch < NRC)
        def _():
            chunk(ch)


@functools.partial(
    pl.kernel,
    out_type=(jax.ShapeDtypeStruct((N, HH), jnp.float32),
              jax.ShapeDtypeStruct((N, HH), jnp.float32)),
    mesh=_mesh,
    scratch_types=(
        [pltpu.VMEM((3, K), jnp.int32) for _ in range(4)]
        + [pltpu.VMEM((K, HH), jnp.float32) for _ in range(4)]
        + [
            pltpu.VMEM_SHARED((216, HH), jnp.float32),
            pltpu.VMEM_SHARED((AGG_R, HH), jnp.float32),
        ]
        + [pltpu.SemaphoreType.DMA for _ in range(6)]
    ),
)
def _edge_kernel(eidx_hbm, h0_hbm, h1_hbm, ccat_hbm,
                 agg0_hbm, agg1_hbm,
                 ib0, ib1, ib2, ib3,
                 hbuf0, hbuf1, cbuf0, cbuf1, ctabS, aggS,
                 semh0, semh1, semc0, semc1, semx0, semx1):
    cid = lax.axis_index("c")
    sid = lax.axis_index("s")
    ib = (ib0, ib1, ib2, ib3)
    hbuf = (hbuf0, hbuf1)
    cbuf = (cbuf0, cbuf1)
    semh = (semh0, semh1)
    semc = (semc0, semc1)
    semx = (semx0, semx1)

    @pl.when(sid == 0)
    def _():
        pltpu.sync_copy(ccat_hbm.at[cid], ctabS)

    _zero_vmem(hbuf0, ZR)
    for k in range(16):
        ch = sid + NS * k

        @pl.when(ch < NZC)
        def _():
            pltpu.sync_copy(hbuf0.at[pl.ds(0, ZR)], aggS.at[pl.ds(ch * ZR, ZR)])

    plsc.subcore_barrier()

    cbase = sid * NCH

    def idxfetch(i, q, x):
        pltpu.async_copy(eidx_hbm.at[cbase + i], ib[q], semx[x])

    def idxwait(q, x):
        pltpu.make_async_copy(eidx_hbm.at[0], ib[q], semx[x]).wait()

    def gissue(q, b):
        pltpu.async_copy(ctabS.at[ib[q].at[1]], cbuf[b], semc[b])

        @pl.when(cid == 0)
        def _():
            pltpu.async_copy(h0_hbm.at[ib[q].at[0]], hbuf[b], semh[b])

        @pl.when(cid == 1)
        def _():
            pltpu.async_copy(h1_hbm.at[ib[q].at[0]], hbuf[b], semh[b])

    idxfetch(0, 0, 0)
    idxfetch(1, 1, 1)
    idxwait(0, 0)
    gissue(0, 0)
    idxwait(1, 1)
    gissue(1, 1)
    idxfetch(2, 2, 0)
    idxfetch(3, 3, 1)

    def halfstep(i, q, b):
        pltpu.make_async_copy(h0_hbm.at[ib[q].at[0]], hbuf[b], semh[b]).wait()
        pltpu.make_async_copy(h0_hbm.at[ib[q].at[0]], cbuf[b], semc[b]).wait()

        def rowbody(r, _):
            for v in range(HH // 16):
                s = pl.ds(v * 16, 16)
                hbuf[b][r, s] = jnp.maximum(hbuf[b][r, s] + cbuf[b][r, s], 0.0)
            return 0

        lax.fori_loop(0, K, rowbody, 0)
        pltpu.sync_copy(hbuf[b], aggS.at[ib[q].at[2]], add=True)

        @pl.when(i + 2 < NCH)
        def _():
            idxwait((q + 2) % 4, b)
            gissue((q + 2) % 4, b)

        @pl.when(i + 4 < NCH)
        def _():
            idxfetch(i + 4, q, b)

    def pipestep(g, _):
        for j in range(4):
            halfstep(4 * g + j, j, j % 2)
        return 0

    lax.fori_loop(0, NCH // 4, pipestep, 0)
    plsc.subcore_barrier()

    for k in range(8):
        ch = sid + NS * k

        @pl.when(ch < NRC)
        def _():
            wb = ch * RZ

            @pl.when(cid == 0)
            def _():
                pltpu.sync_copy(aggS.at[pl.ds(wb, RZ)], agg0_hbm.at[pl.ds(wb, RZ)])

            @pl.when(cid == 1)
            def _():
                pltpu.sync_copy(aggS.at[pl.ds(wb, RZ)], agg1_hbm.at[pl.ds(wb, RZ)])


def _mlp_body(h0r, h1r, a0r, a1r, w1r, b1r, w2r, b2r, z2r, sumr, sqr):
    i = pl.program_id(0)
    z = jnp.concatenate([h0r[...] + a0r[...], h1r[...] + a1r[...]], axis=1)
    z1 = jnp.maximum(
        jnp.dot(z, w1r[...], preferred_element_type=jnp.float32) + b1r[...], 0.0)
    z2 = jnp.dot(z1, w2r[...], preferred_element_type=jnp.float32) + b2r[...]
    z2r[...] = z2
    p = z2.reshape(MB // 8, 8, H)
    s8 = jnp.sum(p, axis=0)
    q8 = jnp.sum(p * p, axis=0)

    @pl.when(i == 0)
    def _():
        sumr[...] = s8
        sqr[...] = q8

    @pl.when(i != 0)
    def _():
        sumr[...] = sumr[...] + s8
        sqr[...] = sqr[...] + q8


_mlp_call = pl.pallas_call(
    _mlp_body,
    grid=(NBLK,),
    in_specs=[
        pl.BlockSpec((MB, HH), lambda i: (i, 0)),
        pl.BlockSpec((MB, HH), lambda i: (i, 0)),
        pl.BlockSpec((MB, HH), lambda i: (i, 0)),
        pl.BlockSpec((MB, HH), lambda i: (i, 0)),
        pl.BlockSpec((H, H), lambda i: (0, 0)),
        pl.BlockSpec((1, H), lambda i: (0, 0)),
        pl.BlockSpec((H, H), lambda i: (0, 0)),
        pl.BlockSpec((1, H), lambda i: (0, 0)),
    ],
    out_specs=[
        pl.BlockSpec((MB, H), lambda i: (i, 0)),
        pl.BlockSpec((8, H), lambda i: (0, 0)),
        pl.BlockSpec((8, H), lambda i: (0, 0)),
    ],
    out_shape=[
        jax.ShapeDtypeStruct((N, H), jnp.float32),
        jax.ShapeDtypeStruct((8, H), jnp.float32),
        jax.ShapeDtypeStruct((8, H), jnp.float32),
    ],
)


def _bnpool_body(z2r, sumr, sqr, ohr, gsumr, cntr):
    i = pl.program_id(0)
    mu = jnp.sum(sumr[...], axis=0, keepdims=True) * (1.0 / N)
    ex2 = jnp.sum(sqr[...], axis=0, keepdims=True) * (1.0 / N)
    var = ex2 - mu * mu
    inv = lax.rsqrt(var + 1e-5)
    hh = jnp.maximum((z2r[...] - mu) * inv, 0.0)
    oh = ohr[...]
    gs = lax.dot_general(oh, hh, (((0,), (0,)), ((), ())),
                         precision=lax.Precision.HIGHEST,
                         preferred_element_type=jnp.float32)
    c8 = jnp.sum(oh.reshape(MB // 8, 8, G), axis=0)

    @pl.when(i == 0)
    def _():
        gsumr[...] = gs
        cntr[...] = c8

    @pl.when(i != 0)
    def _():
        gsumr[...] = gsumr[...] + gs
        cntr[...] = cntr[...] + c8


_bnpool_call = pl.pallas_call(
    _bnpool_body,
    grid=(NBLK,),
    in_specs=[
        pl.BlockSpec((MB, H), lambda i: (i, 0)),
        pl.BlockSpec((8, H), lambda i: (0, 0)),
        pl.BlockSpec((8, H), lambda i: (0, 0)),
        pl.BlockSpec((MB, G), lambda i: (i, 0)),
    ],
    out_specs=[
        pl.BlockSpec((G, H), lambda i: (0, 0)),
        pl.BlockSpec((8, G), lambda i: (0, 0)),
    ],
    out_shape=[
        jax.ShapeDtypeStruct((G, H), jnp.float32),
        jax.ShapeDtypeStruct((8, G), jnp.float32),
    ],
)


def _bn_body(z2r, sumr, sqr, h0r, h1r):
    mu = jnp.sum(sumr[...], axis=0, keepdims=True) * (1.0 / N)
    ex2 = jnp.sum(sqr[...], axis=0, keepdims=True) * (1.0 / N)
    var = ex2 - mu * mu
    inv = lax.rsqrt(var + 1e-5)
    hh = jnp.maximum((z2r[...] - mu) * inv, 0.0)
    h0r[...] = hh[:, :HH]
    h1r[...] = hh[:, HH:]


_bn_call = pl.pallas_call(
    _bn_body,
    grid=(NBLK,),
    in_specs=[
        pl.BlockSpec((MB, H), lambda i: (i, 0)),
        pl.BlockSpec((8, H), lambda i: (0, 0)),
        pl.BlockSpec((8, H), lambda i: (0, 0)),
    ],
    out_specs=[
        pl.BlockSpec((MB, HH), lambda i: (i, 0)),
        pl.BlockSpec((MB, HH), lambda i: (i, 0)),
    ],
    out_shape=[
        jax.ShapeDtypeStruct((N, HH), jnp.float32),
        jax.ShapeDtypeStruct((N, HH), jnp.float32),
    ],
)


def _out_body(gsr, cntr, wo1r, bo1r, wo2r, bo2r, outr):
    c = jnp.maximum(jnp.sum(cntr[...], axis=0, keepdims=True), 1.0)
    g = gsr[...] / c.T
    q = jnp.maximum(
        jnp.dot(g, wo1r[...], preferred_element_type=jnp.float32) + bo1r[...], 0.0)
    outr[...] = jnp.dot(q, wo2r[...], preferred_element_type=jnp.float32) + bo2r[...]


_out_call = pl.pallas_call(
    _out_body,
    grid=(1,),
    in_specs=[
        pl.BlockSpec((G, H), lambda i: (0, 0)),
        pl.BlockSpec((8, G), lambda i: (0, 0)),
        pl.BlockSpec((H, H), lambda i: (0, 0)),
        pl.BlockSpec((1, H), lambda i: (0, 0)),
        pl.BlockSpec((H, OUT), lambda i: (0, 0)),
        pl.BlockSpec((1, OUT), lambda i: (0, 0)),
    ],
    out_specs=pl.BlockSpec((G, OUT), lambda i: (0, 0)),
    out_shape=jax.ShapeDtypeStruct((G, OUT), jnp.float32),
)


def kernel(x, edge_index, edge_attr, batch, atom_emb, bond_emb,
           W1, b1, W2, b2, Wo1, bo1, Wo2, bo2):
    x = x.astype(jnp.int32)
    xoff = (x.T + (jnp.arange(9, dtype=jnp.int32) * 100)[:, None])
    xoffr = xoff.reshape(9, NRC, RZ).transpose(1, 0, 2)
    acat = jnp.concatenate(
        [atom_emb[:, :, :HH].reshape(900, HH),
         atom_emb[:, :, HH:].reshape(900, HH)], axis=0)

    src = edge_index[0].astype(jnp.int32)
    dst = edge_index[1].astype(jnp.int32)
    ea = edge_attr.astype(jnp.int32)
    cidx = ea[:, 0] * 36 + ea[:, 1] * 6 + ea[:, 2]
    pad = EPAD - E
    src_p = jnp.concatenate([src, jnp.zeros((pad,), jnp.int32)])
    dst_p = jnp.concatenate([dst, jnp.full((pad,), N, jnp.int32)])
    c_p = jnp.concatenate([cidx, jnp.zeros((pad,), jnp.int32)])
    eidx = jnp.stack([src_p.reshape(NS, NCH, K), c_p.reshape(NS, NCH, K),
                      dst_p.reshape(NS, NCH, K)], axis=2).reshape(NS * NCH, 3, K)

    C = (bond_emb[:, 0, :, None, None, :]
         + bond_emb[:, 1, None, :, None, :]
         + bond_emb[:, 2, None, None, :, :]).reshape(NL, 216, H)
    ccat = jnp.concatenate([C[:, :, :HH], C[:, :, HH:]], axis=1)
    ccat = ccat.reshape(NL, 2, 216, HH)

    batch_oh = (batch.astype(jnp.int32)[:, None]
                == jnp.arange(G, dtype=jnp.int32)[None, :]).astype(jnp.float32)

    h0, h1 = _atom_kernel(xoffr, acat)
    for l in range(NL):
        agg0, agg1 = _edge_kernel(eidx, h0, h1, ccat[l])
        z2, s8, q8 = _mlp_call(h0, h1, agg0, agg1, W1[l],
                               b1[l].reshape(1, H), W2[l], b2[l].reshape(1, H))
        if l < NL - 1:
            h0, h1 = _bn_call(z2, s8, q8)
        else:
            gsum, cnt8 = _bnpool_call(z2, s8, q8, batch_oh)
    out = _out_call(gsum, cnt8, Wo1, bo1.reshape(1, H),
                    Wo2, bo2.reshape(1, OUT))
    return out
